# Initial kernel scaffold; baseline (speedup 1.0000x reference)
import functools

import jax
import jax.numpy as jnp
from jax import lax
from jax.experimental import pallas as pl
from jax.experimental.pallas import tpu as pltpu
from jax.experimental.pallas import tpu_sc as plsc

_VOCAB = 1000000
_EMBED_DIM = 32
_B_TOTAL = 16384 * 50

_NUM_WORKERS = 32
_B_PER_W = _B_TOTAL // _NUM_WORKERS
_CHUNK = 1600
_N_CHUNKS = _B_PER_W // _CHUNK


def _make_lookup():
  mesh = plsc.VectorSubcoreMesh(core_axis_name="c", subcore_axis_name="s")

  @functools.partial(
      pl.kernel,
      mesh=mesh,
      out_type=jax.ShapeDtypeStruct((_B_TOTAL, _EMBED_DIM), jnp.float32),
      scratch_types=[
          pltpu.VMEM((_CHUNK,), jnp.int32),
          pltpu.VMEM((_CHUNK, _EMBED_DIM), jnp.float32),
          pltpu.SemaphoreType.DMA,
      ],
  )
  def lookup(idx_hbm, table_hbm, out_hbm, idx_v, rows_v, sem):
    wid = lax.axis_index("s") * 2 + lax.axis_index("c")
    base = wid * _B_PER_W

    def body(i, carry):
      off = base + i * _CHUNK
      pltpu.sync_copy(idx_hbm.at[pl.ds(off, _CHUNK)], idx_v)
      pltpu.async_copy(table_hbm.at[idx_v], rows_v, sem).wait()
      pltpu.sync_copy(rows_v, out_hbm.at[pl.ds(off, _CHUNK)])
      return carry

    lax.fori_loop(0, _N_CHUNKS, body, 0)

  return lookup


_lookup = _make_lookup()


@jax.jit
def kernel(inputs, embeddings):
  idx_flat = jnp.reshape(inputs, (_B_TOTAL,)).astype(jnp.int32)
  out = _lookup(idx_flat, embeddings)
  return jnp.reshape(out, (*inputs.shape, _EMBED_DIM))


# speedup vs baseline: 1.1030x; 1.1030x over previous
import functools

import jax
import jax.numpy as jnp
from jax import lax
from jax.experimental import pallas as pl
from jax.experimental.pallas import tpu as pltpu
from jax.experimental.pallas import tpu_sc as plsc

_VOCAB = 1000000
_EMBED_DIM = 32
_B_TOTAL = 16384 * 50

_NUM_WORKERS = 32
_B_PER_W = _B_TOTAL // _NUM_WORKERS
_CHUNK = 1600
_N_CHUNKS = _B_PER_W // _CHUNK


def _make_lookup():
  mesh = plsc.VectorSubcoreMesh(core_axis_name="c", subcore_axis_name="s")

  @functools.partial(
      pl.kernel,
      mesh=mesh,
      out_type=jax.ShapeDtypeStruct((_B_TOTAL, _EMBED_DIM), jnp.float32),
      scratch_types=[
          pltpu.VMEM((_CHUNK,), jnp.int32),
          pltpu.VMEM((_CHUNK, _EMBED_DIM), jnp.float32),
          pltpu.SemaphoreType.DMA,
      ],
      compiler_params=pltpu.CompilerParams(use_tc_tiling_on_sc=False),
  )
  def lookup(idx_hbm, table_hbm, out_hbm, idx_v, rows_v, sem):
    wid = lax.axis_index("s") * 2 + lax.axis_index("c")
    base = wid * _B_PER_W

    def body(i, carry):
      off = base + i * _CHUNK
      pltpu.sync_copy(idx_hbm.at[pl.ds(off, _CHUNK)], idx_v)
      pltpu.async_copy(table_hbm.at[idx_v], rows_v, sem).wait()
      pltpu.sync_copy(rows_v, out_hbm.at[pl.ds(off, _CHUNK)])
      return carry

    lax.fori_loop(0, _N_CHUNKS, body, 0)

  return lookup


_lookup = _make_lookup()


@jax.jit
def kernel(inputs, embeddings):
  idx_flat = jnp.reshape(inputs, (_B_TOTAL,)).astype(jnp.int32)
  out = _lookup(idx_flat, embeddings)
  return jnp.reshape(out, (*inputs.shape, _EMBED_DIM))


# speedup vs baseline: 1.1127x; 1.0088x over previous
import functools

import jax
import jax.numpy as jnp
from jax import lax
from jax.experimental import pallas as pl
from jax.experimental.pallas import tpu as pltpu
from jax.experimental.pallas import tpu_sc as plsc

_VOCAB = 1000000
_EMBED_DIM = 32
_B_TOTAL = 16384 * 50

_NUM_WORKERS = 32
_B_PER_W = _B_TOTAL // _NUM_WORKERS
_CHUNK = 800
_N_CHUNKS = _B_PER_W // _CHUNK
_NBUF = 4


def _make_lookup():
  mesh = plsc.VectorSubcoreMesh(core_axis_name="c", subcore_axis_name="s")

  @functools.partial(
      pl.kernel,
      mesh=mesh,
      out_type=jax.ShapeDtypeStruct((_B_TOTAL, _EMBED_DIM), jnp.float32),
      scratch_types=[
          [pltpu.VMEM((_CHUNK,), jnp.int32)] * _NBUF,
          [pltpu.VMEM((_CHUNK, _EMBED_DIM), jnp.float32)] * _NBUF,
          [pltpu.SemaphoreType.DMA] * _NBUF,
          [pltpu.SemaphoreType.DMA] * _NBUF,
      ],
      compiler_params=pltpu.CompilerParams(use_tc_tiling_on_sc=False),
  )
  def lookup(idx_hbm, table_hbm, out_hbm, idxs, rows, gsems, ssems):
    wid = lax.axis_index("s") * 2 + lax.axis_index("c")
    base = wid * _B_PER_W

    gathers = [None] * _NBUF
    stores = [None] * _NBUF
    for b in range(_NBUF):
      pltpu.sync_copy(idx_hbm.at[pl.ds(base + b * _CHUNK, _CHUNK)], idxs[b])
      gathers[b] = pltpu.async_copy(table_hbm.at[idxs[b]], rows[b], gsems[b])
    for i in range(_N_CHUNKS):
      b = i % _NBUF
      gathers[b].wait()
      stores[b] = pltpu.async_copy(
          rows[b], out_hbm.at[pl.ds(base + i * _CHUNK, _CHUNK)], ssems[b])
      nxt = i + _NBUF
      if nxt < _N_CHUNKS:
        pltpu.sync_copy(idx_hbm.at[pl.ds(base + nxt * _CHUNK, _CHUNK)],
                        idxs[b])
        stores[b].wait()
        gathers[b] = pltpu.async_copy(table_hbm.at[idxs[b]], rows[b],
                                      gsems[b])
    for b in range(_NBUF):
      stores[b].wait()

  return lookup


_lookup = _make_lookup()


@jax.jit
def kernel(inputs, embeddings):
  idx_flat = jnp.reshape(inputs, (_B_TOTAL,)).astype(jnp.int32)
  out = _lookup(idx_flat, embeddings)
  return jnp.reshape(out, (*inputs.shape, _EMBED_DIM))


# speedup vs baseline: 1.4644x; 1.3160x over previous
import functools

import jax
import jax.numpy as jnp
from jax import lax
from jax.experimental import pallas as pl
from jax.experimental.pallas import tpu as pltpu
from jax.experimental.pallas import tpu_sc as plsc

_VOCAB = 1000000
_D = 32
_B = 16384
_H = 50

_NUM_WORKERS = 32
_BW = _B // _NUM_WORKERS
_LANES = 16


def _make_lookup():
  mesh = plsc.VectorSubcoreMesh(core_axis_name="c", subcore_axis_name="s")

  @functools.partial(
      pl.kernel,
      mesh=mesh,
      out_type=jax.ShapeDtypeStruct((_H, _D, _B), jnp.float32),
      scratch_types=[
          [pltpu.VMEM((_BW,), jnp.int32)] * 2,
          [pltpu.VMEM((_BW, _D), jnp.float32)] * 2,
          [pltpu.VMEM((_D, _BW), jnp.float32)] * 2,
          [pltpu.SemaphoreType.DMA] * 2,
          [pltpu.SemaphoreType.DMA] * 2,
      ],
      compiler_params=pltpu.CompilerParams(
          use_tc_tiling_on_sc=False, needs_layout_passes=False),
  )
  def lookup(idx_hbm, table_hbm, out_hbm, idxs, rows, rowts, gsems, wsems):
    wid = lax.axis_index("s") * 2 + lax.axis_index("c")
    b0 = wid * _BW
    iot = lax.iota(jnp.int32, 16)

    def gstart(par, h):
      pltpu.sync_copy(idx_hbm.at[pl.ds(h * _B + b0, _BW)], idxs[par])
      pltpu.async_copy(table_hbm.at[idxs[par]], rows[par], gsems[par])

    def gwait(par):
      pltpu.make_async_copy(table_hbm.at[idxs[par]], rows[par],
                            gsems[par]).wait()

    def wstart(par, h):
      pltpu.async_copy(rowts[par], out_hbm.at[h, :, pl.ds(b0, _BW)],
                       wsems[par])

    def wwait(par):
      pltpu.make_async_copy(rowts[par], out_hbm.at[0, :, pl.ds(b0, _BW)],
                            wsems[par]).wait()

    def transpose(par):
      r = rows[par]
      rt = rowts[par]

      def dbody(d, carry):
        col = jnp.zeros((16,), jnp.int32) + d
        for kb in range(_BW // _LANES):
          ridx = iot + (kb * _LANES)
          rt[d, pl.ds(kb * _LANES, _LANES)] = plsc.load_gather(
              r, [ridx, col])
        return carry

      lax.fori_loop(0, _D, dbody, 0)

    gstart(0, 0)
    gstart(1, 1)

    def step(k, carry):
      for par in (0, 1):
        h = 2 * k + par
        gwait(par)

        @pl.when(h >= 2)
        def _():
          wwait(par)

        transpose(par)
        wstart(par, h)

        @pl.when(h + 2 < _H)
        def _():
          gstart(par, h + 2)

      return carry

    lax.fori_loop(0, _H // 2, step, 0)
    wwait(0)
    wwait(1)

  return lookup


_lookup = _make_lookup()


@jax.jit
def kernel(inputs, embeddings):
  idx_hm = jnp.reshape(jnp.transpose(inputs, (1, 0)), (_H * _B,)).astype(
      jnp.int32)
  out_hdb = _lookup(idx_hm, embeddings)
  return jnp.transpose(out_hdb, (2, 0, 1))


# speedup vs baseline: 1.4684x; 1.0028x over previous
import functools

import jax
import jax.numpy as jnp
from jax import lax
from jax.experimental import pallas as pl
from jax.experimental.pallas import tpu as pltpu
from jax.experimental.pallas import tpu_sc as plsc

_VOCAB = 1000000
_D = 32
_B = 16384
_H = 50

_NUM_WORKERS = 32
_BW = _B // _NUM_WORKERS
_LANES = 16


def _make_lookup():
  mesh = plsc.VectorSubcoreMesh(core_axis_name="c", subcore_axis_name="s")

  @functools.partial(
      pl.kernel,
      mesh=mesh,
      out_type=jax.ShapeDtypeStruct((_H, _D, _B), jnp.float32),
      scratch_types=[
          [pltpu.VMEM((_BW,), jnp.int32)] * 2,
          [pltpu.VMEM((_BW, _D), jnp.float32)] * 2,
          [pltpu.VMEM((_D, _BW), jnp.float32)] * 2,
          [pltpu.SemaphoreType.DMA] * 2,
          [pltpu.SemaphoreType.DMA] * 2,
      ],
      compiler_params=pltpu.CompilerParams(
          use_tc_tiling_on_sc=False, needs_layout_passes=False),
  )
  def lookup(idx_hbm, table_hbm, out_hbm, idxs, rows, rowts, gsems, wsems):
    wid = lax.axis_index("s") * 2 + lax.axis_index("c")
    b0 = wid * _BW
    iot = lax.iota(jnp.int32, 16)

    def gstart(par, h):
      pltpu.sync_copy(idx_hbm.at[pl.ds(h * _B + b0, _BW)], idxs[par])
      pltpu.async_copy(table_hbm.at[idxs[par]], rows[par], gsems[par])

    def gwait(par):
      pltpu.make_async_copy(table_hbm.at[idxs[par]], rows[par],
                            gsems[par]).wait()

    def wstart(par, h):
      pltpu.async_copy(rowts[par], out_hbm.at[h, :, pl.ds(b0, _BW)],
                       wsems[par])

    def wwait(par):
      pltpu.make_async_copy(rowts[par], out_hbm.at[0, :, pl.ds(b0, _BW)],
                            wsems[par]).wait()

    ridxs = [iot + kb * _LANES for kb in range(_BW // _LANES)]

    def transpose(par):
      r = rows[par]
      rt = rowts[par]

      def dbody(d4, carry):
        for dd in range(4):
          d = d4 * 4 + dd
          col = jnp.zeros((16,), jnp.int32) + d
          for kb in range(_BW // _LANES):
            rt[d, pl.ds(kb * _LANES, _LANES)] = plsc.load_gather(
                r, [ridxs[kb], col])
        return carry

      lax.fori_loop(0, _D // 4, dbody, 0)

    gstart(0, 0)
    gstart(1, 1)

    def step(k, carry):
      for par in (0, 1):
        h = 2 * k + par
        gwait(par)

        @pl.when(h >= 2)
        def _():
          wwait(par)

        transpose(par)
        wstart(par, h)

        @pl.when(h + 2 < _H)
        def _():
          gstart(par, h + 2)

      return carry

    lax.fori_loop(0, _H // 2, step, 0)
    wwait(0)
    wwait(1)

  return lookup


_lookup = _make_lookup()


@jax.jit
def kernel(inputs, embeddings):
  idx_hm = jnp.reshape(jnp.transpose(inputs, (1, 0)), (_H * _B,)).astype(
      jnp.int32)
  out_hdb = _lookup(idx_hm, embeddings)
  return jnp.transpose(out_hdb, (2, 0, 1))


# speedup vs baseline: 2.2080x; 1.5037x over previous
import functools

import jax
import jax.numpy as jnp
from jax import lax
from jax.experimental import pallas as pl
from jax.experimental.pallas import tpu as pltpu
from jax.experimental.pallas import tpu_sc as plsc

_VOCAB = 1000000
_D = 32
_B = 16384
_H = 50

_NUM_WORKERS = 32
_BW = _B // _NUM_WORKERS
_LANES = 16


def _make_lookup():
  mesh = plsc.VectorSubcoreMesh(core_axis_name="c", subcore_axis_name="s")

  @functools.partial(
      pl.kernel,
      mesh=mesh,
      out_type=jax.ShapeDtypeStruct((_H, _D, _B), jnp.float32),
      scratch_types=[
          [pltpu.VMEM((_BW,), jnp.int32)] * 2,
          [pltpu.VMEM((_BW, _D), jnp.float32)] * 2,
          [pltpu.VMEM((_D, _BW + 4), jnp.float32)] * 2,
          [pltpu.SemaphoreType.DMA] * 2,
          [pltpu.SemaphoreType.DMA] * 2,
      ],
      compiler_params=pltpu.CompilerParams(
          use_tc_tiling_on_sc=False, needs_layout_passes=False),
  )
  def lookup(idx_hbm, table_hbm, out_hbm, idxs, rows, rowts, gsems, wsems):
    wid = lax.axis_index("s") * 2 + lax.axis_index("c")
    b0 = wid * _BW
    iot = lax.iota(jnp.int32, 16)

    def gstart(par, h):
      pltpu.sync_copy(idx_hbm.at[pl.ds(h * _B + b0, _BW)], idxs[par])
      pltpu.async_copy(table_hbm.at[idxs[par]], rows[par], gsems[par])

    def gwait(par):
      pltpu.make_async_copy(table_hbm.at[idxs[par]], rows[par],
                            gsems[par]).wait()

    def wstart(par, h):
      pltpu.async_copy(rowts[par].at[:, pl.ds(0, _BW)],
                       out_hbm.at[h, :, pl.ds(b0, _BW)], wsems[par])

    def wwait(par):
      pltpu.make_async_copy(rowts[par].at[:, pl.ds(0, _BW)],
                            out_hbm.at[0, :, pl.ds(b0, _BW)],
                            wsems[par]).wait()

    didx = (lax.iota(jnp.int32, 16), lax.iota(jnp.int32, 16) + 16)

    def transpose(par):
      r = rows[par]
      rt = rowts[par]

      def bbody(b8, carry):
        for bb in range(8):
          b = b8 * 8 + bb
          colb = jnp.zeros((16,), jnp.int32) + b
          for dh in range(2):
            plsc.store_scatter(rt, [didx[dh], colb],
                               r[b, pl.ds(dh * _LANES, _LANES)])
        return carry

      lax.fori_loop(0, _BW // 8, bbody, 0)

    gstart(0, 0)
    gstart(1, 1)

    def step(k, carry):
      for par in (0, 1):
        h = 2 * k + par
        gwait(par)

        @pl.when(h >= 2)
        def _():
          wwait(par)

        transpose(par)
        wstart(par, h)

        @pl.when(h + 2 < _H)
        def _():
          gstart(par, h + 2)

      return carry

    lax.fori_loop(0, _H // 2, step, 0)
    wwait(0)
    wwait(1)

  return lookup


_lookup = _make_lookup()


@jax.jit
def kernel(inputs, embeddings):
  idx_hm = jnp.reshape(jnp.transpose(inputs, (1, 0)), (_H * _B,)).astype(
      jnp.int32)
  out_hdb = _lookup(idx_hm, embeddings)
  return jnp.transpose(out_hdb, (2, 0, 1))


# speedup vs baseline: 2.3017x; 1.0424x over previous
import functools

import jax
import jax.numpy as jnp
from jax import lax
from jax.experimental import pallas as pl
from jax.experimental.pallas import tpu as pltpu
from jax.experimental.pallas import tpu_sc as plsc

_VOCAB = 1000000
_D = 32
_B = 16384
_H = 50

_NUM_WORKERS = 32
_BW = _B // _NUM_WORKERS
_LANES = 16


def _make_lookup():
  mesh = plsc.VectorSubcoreMesh(core_axis_name="c", subcore_axis_name="s")

  @functools.partial(
      pl.kernel,
      mesh=mesh,
      out_type=jax.ShapeDtypeStruct((_H, _D, _B), jnp.float32),
      scratch_types=[
          pltpu.VMEM((_H, _BW), jnp.int32),
          [pltpu.VMEM((_BW, _D), jnp.float32)] * 2,
          [pltpu.VMEM((_D, _BW + 4), jnp.float32)] * 2,
          [pltpu.SemaphoreType.DMA] * 2,
          [pltpu.SemaphoreType.DMA] * 2,
      ],
      compiler_params=pltpu.CompilerParams(
          use_tc_tiling_on_sc=False, needs_layout_passes=False),
  )
  def lookup(idx_hbm, table_hbm, out_hbm, idxv, rows, rowts, gsems, wsems):
    wid = lax.axis_index("s") * 2 + lax.axis_index("c")
    b0 = wid * _BW
    iot = lax.iota(jnp.int32, 16)

    pltpu.sync_copy(idx_hbm.at[:, pl.ds(b0, _BW)], idxv)

    def gstart(par, h):
      pltpu.async_copy(table_hbm.at[idxv.at[h]], rows[par], gsems[par])

    def gwait(par):
      pltpu.make_async_copy(table_hbm.at[idxv.at[0]], rows[par],
                            gsems[par]).wait()

    def wstart(par, h):
      pltpu.async_copy(rowts[par].at[:, pl.ds(0, _BW)],
                       out_hbm.at[h, :, pl.ds(b0, _BW)], wsems[par])

    def wwait(par):
      pltpu.make_async_copy(rowts[par].at[:, pl.ds(0, _BW)],
                            out_hbm.at[0, :, pl.ds(b0, _BW)],
                            wsems[par]).wait()

    didx = (lax.iota(jnp.int32, 16), lax.iota(jnp.int32, 16) + 16)

    def transpose(par):
      r = rows[par]
      rt = rowts[par]

      def bbody(b16, carry):
        for bb in range(16):
          b = b16 * 16 + bb
          colb = jnp.zeros((16,), jnp.int32) + b
          for dh in range(2):
            plsc.store_scatter(rt, [didx[dh], colb],
                               r[b, pl.ds(dh * _LANES, _LANES)])
        return carry

      lax.fori_loop(0, _BW // 16, bbody, 0)

    gstart(0, 0)
    gstart(1, 1)

    def step(k, carry):
      for par in (0, 1):
        h = 2 * k + par
        gwait(par)

        @pl.when(h >= 2)
        def _():
          wwait(par)

        transpose(par)
        wstart(par, h)

        @pl.when(h + 2 < _H)
        def _():
          gstart(par, h + 2)

      return carry

    lax.fori_loop(0, _H // 2, step, 0)
    wwait(0)
    wwait(1)

  return lookup


_lookup = _make_lookup()


@jax.jit
def kernel(inputs, embeddings):
  idx_hm = jnp.transpose(inputs, (1, 0)).astype(jnp.int32)
  out_hdb = _lookup(idx_hm, embeddings)
  return jnp.transpose(out_hdb, (2, 0, 1))
